# Initial kernel scaffold; baseline (speedup 1.0000x reference)
import functools

import jax
import jax.numpy as jnp
from jax import lax
from jax.experimental import pallas as pl
from jax.experimental.pallas import tpu as pltpu
from jax.experimental.pallas import tpu_sc as plsc

N = 10000
E = 320000
D = 128

NC = 2
NS = 16
NW = NC * NS
CHUNK = E // NW
K = 125
NBLK = CHUNK // K
RPT = N // NS
DW = 16

_mesh = plsc.VectorSubcoreMesh(core_axis_name="c", subcore_axis_name="s")


@functools.partial(
    pl.kernel,
    mesh=_mesh,
    out_type=jax.ShapeDtypeStruct((NC, N, DW), jnp.float32),
    scratch_types=[
        pltpu.VMEM((NBLK, K), jnp.int32),
        pltpu.VMEM((K, DW), jnp.float32),
        pltpu.VMEM_SHARED((N, DW), jnp.float32),
    ],
)
def _deg_kernel(col_hbm, ones_hbm, zeros_hbm, out_hbm, cols_v, ones_v, acc_sh):
    cid = lax.axis_index("c")
    sid = lax.axis_index("s")
    wid = sid * NC + cid
    r0 = sid * RPT
    pltpu.sync_copy(zeros_hbm.at[pl.ds(r0, RPT)], acc_sh.at[pl.ds(r0, RPT)])
    pltpu.sync_copy(ones_hbm, ones_v)
    pltpu.sync_copy(col_hbm.at[wid], cols_v)
    plsc.subcore_barrier()

    def body(j, carry):
        pltpu.sync_copy(ones_v, acc_sh.at[cols_v.at[j]], add=True)
        return carry

    lax.fori_loop(0, NBLK, body, 0)
    plsc.subcore_barrier()
    pltpu.sync_copy(acc_sh.at[pl.ds(r0, RPT)], out_hbm.at[cid].at[pl.ds(r0, RPT)])


@functools.partial(
    pl.kernel,
    mesh=_mesh,
    out_type=jax.ShapeDtypeStruct((NC, N, D), jnp.float32),
    scratch_types=[
        pltpu.VMEM((NBLK, K), jnp.int32),
        pltpu.VMEM((NBLK, K), jnp.int32),
        pltpu.VMEM((K, D), jnp.float32),
        pltpu.VMEM_SHARED((N, D), jnp.float32),
        pltpu.SemaphoreType.DMA,
    ],
)
def _scatter_kernel(g_hbm, row_hbm, col_hbm, zeros_hbm, out_hbm,
                    rows_v, cols_v, buf_v, acc_sh, sem):
    cid = lax.axis_index("c")
    sid = lax.axis_index("s")
    wid = sid * NC + cid
    r0 = sid * RPT

    @pl.when(cid == 0)
    def _():
        pltpu.sync_copy(g_hbm.at[pl.ds(r0, RPT)], acc_sh.at[pl.ds(r0, RPT)])

    @pl.when(cid == 1)
    def _():
        pltpu.sync_copy(zeros_hbm.at[pl.ds(r0, RPT)], acc_sh.at[pl.ds(r0, RPT)])

    pltpu.sync_copy(row_hbm.at[wid], rows_v)
    pltpu.sync_copy(col_hbm.at[wid], cols_v)
    plsc.subcore_barrier()

    def body(j, carry):
        pltpu.async_copy(g_hbm.at[rows_v.at[j]], buf_v, sem).wait()
        pltpu.sync_copy(buf_v, acc_sh.at[cols_v.at[j]], add=True)
        return carry

    lax.fori_loop(0, NBLK, body, 0)
    plsc.subcore_barrier()
    pltpu.sync_copy(acc_sh.at[pl.ds(r0, RPT)], out_hbm.at[cid].at[pl.ds(r0, RPT)])


_RB = 1000


def _mm_body(x_ref, w_ref, o_ref):
    o_ref[...] = jnp.dot(x_ref[...], w_ref[...],
                         preferred_element_type=jnp.float32)


def _matmul(x, w):
    return pl.pallas_call(
        _mm_body,
        grid=(N // _RB,),
        in_specs=[
            pl.BlockSpec((_RB, D), lambda i: (i, 0)),
            pl.BlockSpec((D, D), lambda i: (0, 0)),
        ],
        out_specs=pl.BlockSpec((_RB, D), lambda i: (i, 0)),
        out_shape=jax.ShapeDtypeStruct((N, D), jnp.float32),
    )(x, w)


def _scale_body(dp_ref, h_ref, dis_ref, g_ref):
    deg = dp_ref[0] + dp_ref[1] + 1.0
    dis = lax.rsqrt(deg)
    dis_ref[...] = dis
    g_ref[...] = h_ref[...] * dis[:, 0:1]


def _scale(dp, h):
    return pl.pallas_call(
        _scale_body,
        grid=(N // _RB,),
        in_specs=[
            pl.BlockSpec((NC, _RB, DW), lambda i: (0, i, 0)),
            pl.BlockSpec((_RB, D), lambda i: (i, 0)),
        ],
        out_specs=[
            pl.BlockSpec((_RB, DW), lambda i: (i, 0)),
            pl.BlockSpec((_RB, D), lambda i: (i, 0)),
        ],
        out_shape=[
            jax.ShapeDtypeStruct((N, DW), jnp.float32),
            jax.ShapeDtypeStruct((N, D), jnp.float32),
        ],
    )(dp, h)


def _mid_body(p_ref, dis_ref, w_ref, b_ref, g_ref):
    dis = dis_ref[...][:, 0:1]
    o1 = dis * (p_ref[0] + p_ref[1]) + b_ref[...]
    o1 = jnp.maximum(o1, 0.0)
    g_ref[...] = jnp.dot(o1, w_ref[...],
                         preferred_element_type=jnp.float32) * dis


def _mid(p, dis, w, b):
    return pl.pallas_call(
        _mid_body,
        grid=(N // _RB,),
        in_specs=[
            pl.BlockSpec((NC, _RB, D), lambda i: (0, i, 0)),
            pl.BlockSpec((_RB, DW), lambda i: (i, 0)),
            pl.BlockSpec((D, D), lambda i: (0, 0)),
            pl.BlockSpec((1, D), lambda i: (0, 0)),
        ],
        out_specs=pl.BlockSpec((_RB, D), lambda i: (i, 0)),
        out_shape=jax.ShapeDtypeStruct((N, D), jnp.float32),
    )(p, dis, w, b)


def _fin_body(p_ref, dis_ref, b_ref, o_ref):
    dis = dis_ref[...][:, 0:1]
    o_ref[...] = dis * (p_ref[0] + p_ref[1]) + b_ref[...]


def _fin(p, dis, b):
    return pl.pallas_call(
        _fin_body,
        grid=(N // _RB,),
        in_specs=[
            pl.BlockSpec((NC, _RB, D), lambda i: (0, i, 0)),
            pl.BlockSpec((_RB, DW), lambda i: (i, 0)),
            pl.BlockSpec((1, D), lambda i: (0, 0)),
        ],
        out_specs=pl.BlockSpec((_RB, D), lambda i: (i, 0)),
        out_shape=jax.ShapeDtypeStruct((N, D), jnp.float32),
    )(p, dis, b)


@jax.jit
def kernel(x, edge_index, W1, b1, W2, b2):
    row = edge_index[0].reshape(NW, NBLK, K)
    col = edge_index[1].reshape(NW, NBLK, K)
    zeros2d = jnp.zeros((N, D), jnp.float32)
    zeros16 = jnp.zeros((N, DW), jnp.float32)
    ones16 = jnp.ones((K, DW), jnp.float32)

    dp = _deg_kernel(col, ones16, zeros16)
    h1 = _matmul(x, W1)
    dis, g1 = _scale(dp, h1)
    p1 = _scatter_kernel(g1, row, col, zeros2d)
    g2 = _mid(p1, dis, W2, b1.reshape(1, D))
    p2 = _scatter_kernel(g2, row, col, zeros2d)
    return _fin(p2, dis, b2.reshape(1, D))


# speedup vs baseline: 20.0528x; 20.0528x over previous
import functools

import jax
import jax.numpy as jnp
from jax import lax
from jax.experimental import pallas as pl
from jax.experimental.pallas import tpu as pltpu
from jax.experimental.pallas import tpu_sc as plsc

N = 10000
E = 320000
D = 128

NC = 2
NS = 16
NW = NC * NS
CHUNK = E // NW
K = 125
NBLK = CHUNK // K
RPT0 = 624
NTAIL0 = NS * RPT0
NTAIL = N - NTAIL0
DW = 128


def _tiled_copy(src, dst, sid):
    r0 = pl.multiple_of(sid * RPT0, 8)
    pltpu.sync_copy(src.at[pl.ds(r0, RPT0)], dst.at[pl.ds(r0, RPT0)])

    @pl.when(sid == NS - 1)
    def _():
        pltpu.sync_copy(src.at[pl.ds(NTAIL0, NTAIL)], dst.at[pl.ds(NTAIL0, NTAIL)])


@functools.cache
def _deg_kernel():
    mesh = plsc.VectorSubcoreMesh(core_axis_name="c", subcore_axis_name="s")
    return functools.partial(
        pl.kernel,
        mesh=mesh,
        out_type=jax.ShapeDtypeStruct((NC, N, DW), jnp.float32),
        scratch_types=[
            pltpu.VMEM((NBLK, K), jnp.int32),
            pltpu.VMEM((K, DW), jnp.float32),
            pltpu.VMEM_SHARED((N, DW), jnp.float32),
        ],
    )(_deg_body)


def _deg_body(col_hbm, ones_hbm, zeros_hbm, out_hbm, cols_v, ones_v, acc_sh):
    cid = lax.axis_index("c")
    sid = lax.axis_index("s")
    wid = sid * NC + cid
    _tiled_copy(zeros_hbm, acc_sh, sid)
    pltpu.sync_copy(ones_hbm, ones_v)
    pltpu.sync_copy(col_hbm.at[wid], cols_v)
    plsc.subcore_barrier()

    def body(j, carry):
        pltpu.sync_copy(ones_v, acc_sh.at[cols_v.at[j]], add=True)
        return carry

    lax.fori_loop(0, NBLK, body, 0)
    plsc.subcore_barrier()
    _tiled_copy(acc_sh, out_hbm.at[cid], sid)


@functools.cache
def _scatter_kernel():
    mesh = plsc.VectorSubcoreMesh(core_axis_name="c", subcore_axis_name="s")
    return functools.partial(
        pl.kernel,
        mesh=mesh,
        out_type=jax.ShapeDtypeStruct((NC, N, D), jnp.float32),
        scratch_types=[
            pltpu.VMEM((NBLK, K), jnp.int32),
            pltpu.VMEM((NBLK, K), jnp.int32),
            pltpu.VMEM((K, D), jnp.float32),
            pltpu.VMEM_SHARED((N, D), jnp.float32),
            pltpu.SemaphoreType.DMA,
        ],
    )(_scatter_body)


def _scatter_body(g_hbm, row_hbm, col_hbm, zeros_hbm, out_hbm,
                  rows_v, cols_v, buf_v, acc_sh, sem):
    cid = lax.axis_index("c")
    sid = lax.axis_index("s")
    wid = sid * NC + cid

    @pl.when(cid == 0)
    def _():
        _tiled_copy(g_hbm, acc_sh, sid)

    @pl.when(cid == 1)
    def _():
        _tiled_copy(zeros_hbm, acc_sh, sid)

    pltpu.sync_copy(row_hbm.at[wid], rows_v)
    pltpu.sync_copy(col_hbm.at[wid], cols_v)
    plsc.subcore_barrier()

    def body(j, carry):
        pltpu.async_copy(g_hbm.at[rows_v.at[j]], buf_v, sem).wait()
        pltpu.sync_copy(buf_v, acc_sh.at[cols_v.at[j]], add=True)
        return carry

    lax.fori_loop(0, NBLK, body, 0)
    plsc.subcore_barrier()
    _tiled_copy(acc_sh, out_hbm.at[cid], sid)


_RB = 1000


def _mm_body(x_ref, w_ref, o_ref):
    o_ref[...] = jnp.dot(x_ref[...], w_ref[...],
                         preferred_element_type=jnp.float32)


def _matmul(x, w):
    return pl.pallas_call(
        _mm_body,
        grid=(N // _RB,),
        in_specs=[
            pl.BlockSpec((_RB, D), lambda i: (i, 0)),
            pl.BlockSpec((D, D), lambda i: (0, 0)),
        ],
        out_specs=pl.BlockSpec((_RB, D), lambda i: (i, 0)),
        out_shape=jax.ShapeDtypeStruct((N, D), jnp.float32),
    )(x, w)


def _scale_body(dp_ref, h_ref, dis_ref, g_ref):
    deg = dp_ref[0] + dp_ref[1] + 1.0
    dis = lax.rsqrt(deg)
    dis_ref[...] = dis
    g_ref[...] = h_ref[...] * dis[:, 0:1]


def _scale(dp, h):
    return pl.pallas_call(
        _scale_body,
        grid=(N // _RB,),
        in_specs=[
            pl.BlockSpec((NC, _RB, DW), lambda i: (0, i, 0)),
            pl.BlockSpec((_RB, D), lambda i: (i, 0)),
        ],
        out_specs=[
            pl.BlockSpec((_RB, DW), lambda i: (i, 0)),
            pl.BlockSpec((_RB, D), lambda i: (i, 0)),
        ],
        out_shape=[
            jax.ShapeDtypeStruct((N, DW), jnp.float32),
            jax.ShapeDtypeStruct((N, D), jnp.float32),
        ],
    )(dp, h)


def _mid_body(p_ref, dis_ref, w_ref, b_ref, g_ref):
    dis = dis_ref[...][:, 0:1]
    o1 = dis * (p_ref[0] + p_ref[1]) + b_ref[...]
    o1 = jnp.maximum(o1, 0.0)
    g_ref[...] = jnp.dot(o1, w_ref[...],
                         preferred_element_type=jnp.float32) * dis


def _mid(p, dis, w, b):
    return pl.pallas_call(
        _mid_body,
        grid=(N // _RB,),
        in_specs=[
            pl.BlockSpec((NC, _RB, D), lambda i: (0, i, 0)),
            pl.BlockSpec((_RB, DW), lambda i: (i, 0)),
            pl.BlockSpec((D, D), lambda i: (0, 0)),
            pl.BlockSpec((1, D), lambda i: (0, 0)),
        ],
        out_specs=pl.BlockSpec((_RB, D), lambda i: (i, 0)),
        out_shape=jax.ShapeDtypeStruct((N, D), jnp.float32),
    )(p, dis, w, b)


def _fin_body(p_ref, dis_ref, b_ref, o_ref):
    dis = dis_ref[...][:, 0:1]
    o_ref[...] = dis * (p_ref[0] + p_ref[1]) + b_ref[...]


def _fin(p, dis, b):
    return pl.pallas_call(
        _fin_body,
        grid=(N // _RB,),
        in_specs=[
            pl.BlockSpec((NC, _RB, D), lambda i: (0, i, 0)),
            pl.BlockSpec((_RB, DW), lambda i: (i, 0)),
            pl.BlockSpec((1, D), lambda i: (0, 0)),
        ],
        out_specs=pl.BlockSpec((_RB, D), lambda i: (i, 0)),
        out_shape=jax.ShapeDtypeStruct((N, D), jnp.float32),
    )(p, dis, b)


@jax.jit
def kernel(x, edge_index, W1, b1, W2, b2):
    row = edge_index[0].reshape(NW, NBLK, K)
    col = edge_index[1].reshape(NW, NBLK, K)
    zeros2d = jnp.zeros((N, D), jnp.float32)
    zeros16 = jnp.zeros((N, DW), jnp.float32)
    ones16 = jnp.ones((K, DW), jnp.float32)

    dp = _deg_kernel()(col, ones16, zeros16)
    h1 = _matmul(x, W1)
    dis, g1 = _scale(dp, h1)
    p1 = _scatter_kernel()(g1, row, col, zeros2d)
    g2 = _mid(p1, dis, W2, b1.reshape(1, D))
    p2 = _scatter_kernel()(g2, row, col, zeros2d)
    return _fin(p2, dis, b2.reshape(1, D))


# speedup vs baseline: 24.7153x; 1.2325x over previous
import functools

import jax
import jax.numpy as jnp
from jax import lax
from jax.experimental import pallas as pl
from jax.experimental.pallas import tpu as pltpu
from jax.experimental.pallas import tpu_sc as plsc

N = 10000
E = 320000
D = 128

NC = 2
NS = 16
NW = NC * NS
CHUNK = E // NW
K = 125
NBLK = CHUNK // K
RPT0 = 624
NTAIL0 = NS * RPT0
NTAIL = N - NTAIL0
DW = 128


def _tiled_copy(src, dst, sid):
    r0 = pl.multiple_of(sid * RPT0, 8)
    pltpu.sync_copy(src.at[pl.ds(r0, RPT0)], dst.at[pl.ds(r0, RPT0)])

    @pl.when(sid == NS - 1)
    def _():
        pltpu.sync_copy(src.at[pl.ds(NTAIL0, NTAIL)], dst.at[pl.ds(NTAIL0, NTAIL)])


@functools.cache
def _deg_kernel():
    mesh = plsc.VectorSubcoreMesh(core_axis_name="c", subcore_axis_name="s")
    return functools.partial(
        pl.kernel,
        mesh=mesh,
        out_type=jax.ShapeDtypeStruct((NC, N, DW), jnp.float32),
        scratch_types=[
            pltpu.VMEM((NBLK, K), jnp.int32),
            pltpu.VMEM((K, DW), jnp.float32),
            pltpu.VMEM_SHARED((N, DW), jnp.float32),
        ],
    )(_deg_body)


def _deg_body(col_hbm, ones_hbm, zeros_hbm, out_hbm, cols_v, ones_v, acc_sh):
    cid = lax.axis_index("c")
    sid = lax.axis_index("s")
    wid = sid * NC + cid
    _tiled_copy(zeros_hbm, acc_sh, sid)
    pltpu.sync_copy(ones_hbm, ones_v)
    pltpu.sync_copy(col_hbm.at[wid], cols_v)
    plsc.subcore_barrier()

    def body(j, carry):
        pltpu.sync_copy(ones_v, acc_sh.at[cols_v.at[j]], add=True)
        return carry

    lax.fori_loop(0, NBLK, body, 0)
    plsc.subcore_barrier()
    _tiled_copy(acc_sh, out_hbm.at[cid], sid)


@functools.cache
def _scatter_kernel():
    mesh = plsc.VectorSubcoreMesh(core_axis_name="c", subcore_axis_name="s")
    return functools.partial(
        pl.kernel,
        mesh=mesh,
        out_type=jax.ShapeDtypeStruct((NC, N, D), jnp.float32),
        scratch_types=[
            pltpu.VMEM((2, K), jnp.int32),
            pltpu.VMEM((2, K), jnp.int32),
            pltpu.VMEM((2, K, D), jnp.float32),
            pltpu.VMEM_SHARED((N, D), jnp.float32),
            pltpu.SemaphoreType.DMA,
            pltpu.SemaphoreType.DMA,
        ],
    )(_scatter_body)


def _scatter_body(g_hbm, row_hbm, col_hbm, zeros_hbm, out_hbm,
                  rows_v, cols_v, buf_v, acc_sh, gsem, isem):
    cid = lax.axis_index("c")
    sid = lax.axis_index("s")
    wid = sid * NC + cid

    @pl.when(cid == 0)
    def _():
        _tiled_copy(g_hbm, acc_sh, sid)

    @pl.when(cid == 1)
    def _():
        _tiled_copy(zeros_hbm, acc_sh, sid)

    plsc.subcore_barrier()

    pltpu.sync_copy(row_hbm.at[wid].at[0], rows_v.at[0])
    pltpu.sync_copy(col_hbm.at[wid].at[0], cols_v.at[0])
    pltpu.async_copy(g_hbm.at[rows_v.at[0]], buf_v.at[0], gsem)
    pltpu.async_copy(row_hbm.at[wid].at[1], rows_v.at[1], isem)
    pltpu.async_copy(col_hbm.at[wid].at[1], cols_v.at[1], isem)

    def body(j, carry):
        slot = lax.rem(j, 2)
        ns = 1 - slot
        pltpu.make_async_copy(g_hbm.at[rows_v.at[slot]], buf_v.at[slot],
                              gsem).wait()

        @pl.when(j + 1 < NBLK)
        def _():
            pltpu.make_async_copy(row_hbm.at[wid].at[j + 1], rows_v.at[ns],
                                  isem).wait()
            pltpu.make_async_copy(col_hbm.at[wid].at[j + 1], cols_v.at[ns],
                                  isem).wait()
            pltpu.async_copy(g_hbm.at[rows_v.at[ns]], buf_v.at[ns], gsem)

        pltpu.sync_copy(buf_v.at[slot], acc_sh.at[cols_v.at[slot]], add=True)

        @pl.when(j + 2 < NBLK)
        def _():
            pltpu.async_copy(row_hbm.at[wid].at[j + 2], rows_v.at[slot], isem)
            pltpu.async_copy(col_hbm.at[wid].at[j + 2], cols_v.at[slot], isem)

        return carry

    lax.fori_loop(0, NBLK, body, 0)
    plsc.subcore_barrier()
    _tiled_copy(acc_sh, out_hbm.at[cid], sid)


_RB = 1000


def _mm_body(x_ref, w_ref, o_ref):
    o_ref[...] = jnp.dot(x_ref[...], w_ref[...],
                         preferred_element_type=jnp.float32)


def _matmul(x, w):
    return pl.pallas_call(
        _mm_body,
        grid=(N // _RB,),
        in_specs=[
            pl.BlockSpec((_RB, D), lambda i: (i, 0)),
            pl.BlockSpec((D, D), lambda i: (0, 0)),
        ],
        out_specs=pl.BlockSpec((_RB, D), lambda i: (i, 0)),
        out_shape=jax.ShapeDtypeStruct((N, D), jnp.float32),
    )(x, w)


def _scale_body(dp_ref, h_ref, dis_ref, g_ref):
    deg = dp_ref[0] + dp_ref[1] + 1.0
    dis = lax.rsqrt(deg)
    dis_ref[...] = dis
    g_ref[...] = h_ref[...] * dis[:, 0:1]


def _scale(dp, h):
    return pl.pallas_call(
        _scale_body,
        grid=(N // _RB,),
        in_specs=[
            pl.BlockSpec((NC, _RB, DW), lambda i: (0, i, 0)),
            pl.BlockSpec((_RB, D), lambda i: (i, 0)),
        ],
        out_specs=[
            pl.BlockSpec((_RB, DW), lambda i: (i, 0)),
            pl.BlockSpec((_RB, D), lambda i: (i, 0)),
        ],
        out_shape=[
            jax.ShapeDtypeStruct((N, DW), jnp.float32),
            jax.ShapeDtypeStruct((N, D), jnp.float32),
        ],
    )(dp, h)


def _mid_body(p_ref, dis_ref, w_ref, b_ref, g_ref):
    dis = dis_ref[...][:, 0:1]
    o1 = dis * (p_ref[0] + p_ref[1]) + b_ref[...]
    o1 = jnp.maximum(o1, 0.0)
    g_ref[...] = jnp.dot(o1, w_ref[...],
                         preferred_element_type=jnp.float32) * dis


def _mid(p, dis, w, b):
    return pl.pallas_call(
        _mid_body,
        grid=(N // _RB,),
        in_specs=[
            pl.BlockSpec((NC, _RB, D), lambda i: (0, i, 0)),
            pl.BlockSpec((_RB, DW), lambda i: (i, 0)),
            pl.BlockSpec((D, D), lambda i: (0, 0)),
            pl.BlockSpec((1, D), lambda i: (0, 0)),
        ],
        out_specs=pl.BlockSpec((_RB, D), lambda i: (i, 0)),
        out_shape=jax.ShapeDtypeStruct((N, D), jnp.float32),
    )(p, dis, w, b)


def _fin_body(p_ref, dis_ref, b_ref, o_ref):
    dis = dis_ref[...][:, 0:1]
    o_ref[...] = dis * (p_ref[0] + p_ref[1]) + b_ref[...]


def _fin(p, dis, b):
    return pl.pallas_call(
        _fin_body,
        grid=(N // _RB,),
        in_specs=[
            pl.BlockSpec((NC, _RB, D), lambda i: (0, i, 0)),
            pl.BlockSpec((_RB, DW), lambda i: (i, 0)),
            pl.BlockSpec((1, D), lambda i: (0, 0)),
        ],
        out_specs=pl.BlockSpec((_RB, D), lambda i: (i, 0)),
        out_shape=jax.ShapeDtypeStruct((N, D), jnp.float32),
    )(p, dis, b)


@jax.jit
def kernel(x, edge_index, W1, b1, W2, b2):
    row = edge_index[0].reshape(NW, NBLK, K)
    col = edge_index[1].reshape(NW, NBLK, K)
    zeros2d = jnp.zeros((N, D), jnp.float32)
    zeros16 = jnp.zeros((N, DW), jnp.float32)
    ones16 = jnp.ones((K, DW), jnp.float32)

    dp = _deg_kernel()(col, ones16, zeros16)
    h1 = _matmul(x, W1)
    dis, g1 = _scale(dp, h1)
    p1 = _scatter_kernel()(g1, row, col, zeros2d)
    g2 = _mid(p1, dis, W2, b1.reshape(1, D))
    p2 = _scatter_kernel()(g2, row, col, zeros2d)
    return _fin(p2, dis, b2.reshape(1, D))
